# baseline (speedup 1.0000x reference)
import functools

import jax
import jax.numpy as jnp
from jax import lax
from jax.experimental import pallas as pl
from jax.experimental.pallas import tpu as pltpu
from jax.experimental.pallas import tpu_sc as plsc

W, H, N = 1024, 1024, 64
EX, EY, EW, EH = -180.0, -90.0, 360.0, 180.0
PX, PY = 2, 2
KX, KY = 2 * PX + 1, 2 * PY + 1
RPE = KX * KY
B = 16384

NC, NS = 2, 16
NWORK = NC * NS
EPW = B // NWORK
CHUNK = 64
NCHUNK = EPW // CHUNK
CR = CHUNK * RPE
GROUPS = CHUNK // 16


def _body(x_hbm, y_hbm, tab_hbm, out_hbm, xs_v, ys_v, src_v, dst_v, rows_v,
          gsem, ssem):
    wid = lax.axis_index("s") * NC + lax.axis_index("c")
    ebase = wid * EPW
    pltpu.sync_copy(x_hbm.at[pl.ds(ebase, EPW)], xs_v)
    pltpu.sync_copy(y_hbm.at[pl.ds(ebase, EPW)], ys_v)
    lanes25 = lax.iota(jnp.int32, 16) * RPE

    def chunk_body(c, carry):
        for g in range(GROUPS):
            e0 = c * CHUNK + g * 16
            xv = xs_v[pl.ds(e0, 16)]
            yv = ys_v[pl.ds(e0, 16)]
            ixv = ((xv - EX) * (W / EW)).astype(jnp.int32)
            iyv = ((yv - EY) * (H / EH)).astype(jnp.int32)
            ixv = jnp.minimum(jnp.maximum(ixv, 0), W - 1)
            iyv = jnp.minimum(jnp.maximum(iyv, 0), H - 1)
            gxs, gys = [], []
            for o in range(-PX, PX + 1):
                gx = ixv + o
                gx = jnp.where(gx < 0, gx + W, gx)
                gx = jnp.where(gx >= W, gx - W, gx)
                gxs.append(gx * H)
            for o in range(-PY, PY + 1):
                gy = iyv + o
                gy = jnp.where(gy < 0, gy + H, gy)
                gy = jnp.where(gy >= H, gy - H, gy)
                gys.append(gy)
            dst0 = (ebase + e0) * RPE + lanes25
            for i in range(KX):
                for j in range(KY):
                    r = i * KY + j
                    k0 = r * CHUNK + g * 16
                    src_v[pl.ds(k0, 16)] = gxs[i] + gys[j]
                    dst_v[pl.ds(k0, 16)] = dst0 + r
        pltpu.async_copy(tab_hbm.at[src_v], rows_v, gsem).wait()
        pltpu.async_copy(rows_v, out_hbm.at[dst_v], ssem).wait()
        return carry

    lax.fori_loop(0, NCHUNK, chunk_body, 0)


@jax.jit
def kernel(input, weight):
    xs = input[:, 0]
    ys = input[:, 1]
    tab = weight.reshape(W * H, N)
    mesh = plsc.VectorSubcoreMesh(core_axis_name="c", subcore_axis_name="s")
    out = pl.kernel(
        _body,
        mesh=mesh,
        out_type=jax.ShapeDtypeStruct((B * RPE, N), jnp.float32),
        scratch_types=[
            pltpu.VMEM((EPW,), jnp.float32),
            pltpu.VMEM((EPW,), jnp.float32),
            pltpu.VMEM((CR,), jnp.int32),
            pltpu.VMEM((CR,), jnp.int32),
            pltpu.VMEM((CR, N), jnp.float32),
            pltpu.SemaphoreType.DMA,
            pltpu.SemaphoreType.DMA,
        ],
        compiler_params=pltpu.CompilerParams(use_tc_tiling_on_sc=False),
    )(xs, ys, tab)
    return out.reshape(B, KX, KY, N)
